# baseline (speedup 1.0000x reference)
import functools

import jax
import jax.numpy as jnp
from jax import lax
from jax.experimental import pallas as pl
from jax.experimental.pallas import tpu as pltpu
from jax.experimental.pallas import tpu_sc as plsc

_NC = 2
_NS = 16
_NW = _NC * _NS

_LOG1P_COEF = (
    0.0005721672283739987,
    0.9812560175991404,
    -0.3941956109139464,
    0.10584377187810023,
)


def _bce16(x, y):
    a = jnp.abs(x)
    u = jnp.exp(-a)
    p = jnp.full((16,), _LOG1P_COEF[3], jnp.float32)
    for c in (_LOG1P_COEF[2], _LOG1P_COEF[1], _LOG1P_COEF[0]):
        p = p * u + c
    return jnp.maximum(x, 0.0) - x * y + p


def _sc_body(blks, chans_per_w, pred_hbm, gt_hbm, mask_hbm, out_hbm,
             pv0, gv0, mv0, pv1, gv1, mv1, rowv, sems):
    w = lax.axis_index("s") * _NC + lax.axis_index("c")
    lane = lax.iota(jnp.int32, 16)
    zf = jnp.zeros((16,), jnp.float32)
    zi = jnp.zeros((16,), jnp.int32)
    nchunks = chans_per_w * 2
    base_r = w * nchunks
    bufs = ((pv0, gv0, mv0), (pv1, gv1, mv1))

    def issue(t, buf):
        r = base_r + t
        pvb, gvb, mvb = bufs[buf]
        pltpu.async_copy(pred_hbm.at[r], pvb, sems.at[buf, 0])
        pltpu.async_copy(gt_hbm.at[r], gvb, sems.at[buf, 1])
        pltpu.async_copy(mask_hbm.at[r], mvb, sems.at[buf, 2])

    def wait(t, buf):
        r = base_r + t
        pvb, gvb, mvb = bufs[buf]
        pltpu.make_async_copy(pred_hbm.at[r], pvb, sems.at[buf, 0]).wait()
        pltpu.make_async_copy(gt_hbm.at[r], gvb, sems.at[buf, 1]).wait()
        pltpu.make_async_copy(mask_hbm.at[r], mvb, sems.at[buf, 2]).wait()

    def chunk_sums(buf, carry):
        pvb, gvb, mvb = bufs[buf]

        def blk_body(b, carry2):
            a0, a1, a2, a3, ac = carry2
            base = b * 64
            w16 = mvb[pl.ds(b * 16, 16)]
            ac = ac + lax.shift_right_logical(
                w16 * jnp.int32(0x01010101), 24)
            accs = []
            for jj, acc in enumerate((a0, a1, a2, a3)):
                x = pvb[pl.ds(base + jj * 16, 16)]
                y = gvb[pl.ds(base + jj * 16, 16)]
                mf = (lax.shift_right_logical(w16, jj * 8) & 1
                      ).astype(jnp.float32)
                accs.append(acc + _bce16(x, y) * mf)
            return (*accs, ac)

        return plsc.parallel_loop(0, blks, carry=carry, unroll=2)(blk_body)

    issue(0, 0)

    def chan_body(cl, loss_vec):
        t0 = cl * 2
        issue(t0 + 1, 1)
        wait(t0, 0)
        carry = chunk_sums(0, (zf, zf, zf, zf, zi))

        @pl.when(cl + 1 < chans_per_w)
        def _():
            issue(t0 + 2, 0)

        wait(t0 + 1, 1)
        a0, a1, a2, a3, ac = chunk_sums(1, carry)
        s = jnp.sum((a0 + a1) + (a2 + a3))
        c = jnp.sum(ac).astype(jnp.float32)
        lossv = jnp.full((16,), s) / jnp.maximum(jnp.full((16,), c), 1.0)
        return jnp.where(lane == cl, lossv, loss_vec)

    loss_vec = lax.fori_loop(0, chans_per_w, chan_body, zf)
    rowv[...] = loss_vec
    pltpu.sync_copy(rowv, out_hbm.at[w])


def kernel(pred, gt, mask):
    B, C, H, W = pred.shape
    nch = B * C
    hw = H * W
    chunk = hw // 2
    blks = chunk // 64
    chans_per_w = nch // _NW
    assert nch % _NW == 0 and hw % 128 == 0

    pred2 = pred.reshape(nch * 2, chunk)
    gt2 = gt.reshape(nch * 2, chunk)
    mask2 = lax.bitcast_convert_type(
        mask.astype(jnp.uint8).reshape(nch * 2, blks, 4, 16)
        .transpose(0, 1, 3, 2), jnp.int32).reshape(nch * 2, blks * 16)

    mesh = plsc.VectorSubcoreMesh(
        core_axis_name="c", subcore_axis_name="s",
        num_cores=_NC, num_subcores=_NS)

    f = pl.kernel(
        functools.partial(_sc_body, blks, chans_per_w),
        out_type=jax.ShapeDtypeStruct((_NW, 16), jnp.float32),
        mesh=mesh,
        scratch_types=[
            pltpu.VMEM((chunk,), jnp.float32),
            pltpu.VMEM((chunk,), jnp.float32),
            pltpu.VMEM((blks * 16,), jnp.int32),
            pltpu.VMEM((chunk,), jnp.float32),
            pltpu.VMEM((chunk,), jnp.float32),
            pltpu.VMEM((blks * 16,), jnp.int32),
            pltpu.VMEM((16,), jnp.float32),
            pltpu.SemaphoreType.DMA((2, 3)),
        ],
        compiler_params=pltpu.CompilerParams(needs_layout_passes=False),
    )
    out = f(pred2, gt2, mask2)
    return jnp.sum(out) / jnp.float32(nch)
